# Initial kernel scaffold; baseline (speedup 1.0000x reference)
import jax
import jax.numpy as jnp
from jax.experimental import pallas as pl


def kernel(output, target):
    raise NotImplementedError("write your pallas kernel here")


# speedup vs baseline: 20.5348x; 20.5348x over previous
import functools

import jax
import jax.numpy as jnp
from jax import lax
from jax.experimental import pallas as pl
from jax.experimental.pallas import tpu as pltpu
from jax.experimental.pallas import tpu_sc as plsc

NA = 5
NCLS = 20
NHW = 19
NPIX = 361
HALF = 192
OVER = 169
AW = (1.3221, 3.19275, 5.05587, 9.47112, 11.2364)
AH = (1.73145, 4.00944, 8.09892, 4.84053, 10.0071)
THRESH = 0.6
OBJ_SCALE = 5.0
LN2 = 0.6931471805599453
SQRT2 = 1.4142135381698608


def _iota16():
    return lax.iota(jnp.int32, 16)


def _sigmoid(v):
    return 1.0 / (1.0 + jnp.exp(-v))


def _sl1(d):
    a = jnp.abs(d)
    return jnp.where(a < 1.0, 0.5 * a * a, a - 0.5)


def _vlog(v):
    bits = plsc.bitcast(v, jnp.int32)
    e = (bits >> 23) - 127
    m = plsc.bitcast((bits & 0x007FFFFF) | 0x3F800000, jnp.float32)
    big = m >= SQRT2
    m = jnp.where(big, m * 0.5, m)
    ef = e.astype(jnp.float32) + jnp.where(big, 1.0, 0.0)
    s = (m - 1.0) / (m + 1.0)
    z = s * s
    p = s * (2.0 + z * (0.6666666865 + z * (0.4000000059
             + z * (0.2857142985 + z * 0.2222222222))))
    return ef * LN2 + p


def _pick_anchor(bn, vals):
    r = jnp.full((16,), vals[0], jnp.float32)
    for a in range(1, NA):
        r = jnp.where(bn == a, vals[a], r)
    return r


_MESH = plsc.VectorSubcoreMesh(core_axis_name="c", subcore_axis_name="s")

_SCRATCH = [
    pltpu.VMEM((125 * NPIX,), jnp.float32),
    pltpu.VMEM((320,), jnp.float32),
    pltpu.VMEM((960,), jnp.float32),
    pltpu.VMEM((960,), jnp.float32),
    pltpu.VMEM((960,), jnp.float32),
    pltpu.VMEM((960,), jnp.float32),
    pltpu.VMEM((960,), jnp.float32),
    pltpu.VMEM((960,), jnp.float32),
    pltpu.VMEM((960,), jnp.float32),
    pltpu.VMEM((960,), jnp.float32),
    pltpu.VMEM((960,), jnp.float32),
    pltpu.VMEM((64,), jnp.float32),
    pltpu.VMEM((64,), jnp.float32),
    pltpu.VMEM((64,), jnp.float32),
    pltpu.VMEM((64,), jnp.float32),
    pltpu.VMEM((64,), jnp.float32),
    pltpu.VMEM((64,), jnp.float32),
    pltpu.VMEM((64,), jnp.float32),
    pltpu.VMEM((64,), jnp.float32),
    pltpu.VMEM((64,), jnp.int32),
    pltpu.VMEM((64,), jnp.int32),
    pltpu.VMEM((64,), jnp.int32),
    pltpu.VMEM((64,), jnp.int32),
    pltpu.VMEM((64,), jnp.int32),
    pltpu.VMEM((16,), jnp.float32),
    pltpu.SMEM((64,), jnp.float32),
    pltpu.SMEM((64,), jnp.float32),
    pltpu.SMEM((64,), jnp.float32),
    pltpu.SMEM((64,), jnp.float32),
    pltpu.SMEM((64,), jnp.float32),
    pltpu.SMEM((64,), jnp.float32),
    pltpu.SMEM((64,), jnp.float32),
    pltpu.SMEM((64,), jnp.int32),
    pltpu.SMEM((64,), jnp.int32),
]


@functools.partial(
    pl.kernel,
    out_type=jax.ShapeDtypeStruct((32, 16), jnp.float32),
    mesh=_MESH,
    scratch_types=_SCRATCH,
    compiler_params=pltpu.CompilerParams(needs_layout_passes=False),
)
def _region_loss_sc(out3, tgt2, out,
                    raw, tgt,
                    cx1, cx2, cy1, cy2, w1h1, xs, ys, cfs, cur,
                    gxr, gyr, gwr, ghr, txva, tyva, twva, thva,
                    idxa, pixa, vala, bna, clsa, accv,
                    sb2x1, sb2x2, sb2y1, sb2y2, sw2, sh2, swh, sidx, sval):
    c = lax.axis_index("c")
    s = lax.axis_index("s")
    b = s
    h = c
    start = h * OVER
    wid = s * 2 + c

    pltpu.sync_copy(tgt2.at[b, 0], tgt)
    pltpu.sync_copy(out3.at[b, 0], raw)

    nzeros = jnp.int32(0)
    for ci in range(4):
        t16 = _iota16() + 16 * ci
        base5 = t16 * 5
        tc0 = plsc.load_gather(tgt, [base5])
        gx = plsc.load_gather(tgt, [base5 + 1]) * float(NHW)
        gy = plsc.load_gather(tgt, [base5 + 2]) * float(NHW)
        gw = plsc.load_gather(tgt, [base5 + 3]) * float(NHW)
        gh = plsc.load_gather(tgt, [base5 + 4]) * float(NHW)

        zc = jnp.where((t16 < 50) & (gx != 0.0), 0, 1).astype(jnp.int32)
        cz = plsc.cumsum(zc)
        vald = (cz + nzeros) == 0
        nzeros = nzeros + jnp.sum(zc)

        best_iou = jnp.full((16,), -1.0, jnp.float32)
        bn = jnp.zeros((16,), jnp.int32)
        for a in range(NA):
            ca = jnp.minimum(gw, AW[a]) * jnp.minimum(gh, AH[a])
            ai = ca / (AW[a] * AH[a] + gw * gh - ca)
            better = ai > best_iou
            best_iou = jnp.where(better, ai, best_iou)
            bn = jnp.where(better, a, bn)

        gi = gx.astype(jnp.int32)
        gj = gy.astype(jnp.int32)
        awn = _pick_anchor(bn, AW)
        ahn = _pick_anchor(bn, AH)
        pix = gj * NHW + gi

        gws = jnp.where(vald, gw, 0.0)
        ghs = jnp.where(vald, gh, 0.0)
        vx1 = gx - 0.5 * gws
        vx2 = gx + 0.5 * gws
        vy1 = gy - 0.5 * ghs
        vy2 = gy + 0.5 * ghs
        vwh = gws * ghs
        vidx = bn * NPIX + pix
        vvalI = vald.astype(jnp.int32)

        dsl = pl.ds(16 * ci, 16)
        gxr[dsl] = gx
        gyr[dsl] = gy
        gwr[dsl] = gw
        ghr[dsl] = gh
        txva[dsl] = gx - gi.astype(jnp.float32)
        tyva[dsl] = gy - gj.astype(jnp.float32)
        twva[dsl] = _vlog(gw / awn)
        thva[dsl] = _vlog(gh / ahn)
        idxa[dsl] = vidx
        pixa[dsl] = pix
        vala[dsl] = vvalI
        bna[dsl] = bn
        clsa[dsl] = tc0.astype(jnp.int32)

        for k in range(16):
            t = 16 * ci + k
            sb2x1[t] = vx1[k]
            sb2x2[t] = vx2[k]
            sb2y1[t] = vy1[k]
            sb2y2[t] = vy2[k]
            sw2[t] = gws[k]
            sh2[t] = ghs[k]
            swh[t] = vwh[k]
            sidx[t] = vidx[k]
            sval[t] = vvalI[k]

    def prep_body(j, acc):
        p16 = j * 16 + _iota16()
        pg = p16 + start
        gxg = (pg % NHW).astype(jnp.float32)
        gyg = (pg // NHW).astype(jnp.float32)
        mreal = p16 >= h * 23
        col = start + j * 16
        for a in range(NA):
            r0 = a * 25
            xv = _sigmoid(raw[pl.ds((r0 + 0) * NPIX + col, 16)])
            yv = _sigmoid(raw[pl.ds((r0 + 1) * NPIX + col, 16)])
            wv = raw[pl.ds((r0 + 2) * NPIX + col, 16)]
            hv = raw[pl.ds((r0 + 3) * NPIX + col, 16)]
            cfv = _sigmoid(raw[pl.ds((r0 + 4) * NPIX + col, 16)])
            px = xv + gxg
            py = yv + gyg
            pw = jnp.exp(wv) * AW[a]
            ph = jnp.exp(hv) * AH[a]
            dso = pl.ds(a * HALF + j * 16, 16)
            cx1[dso] = px - 0.5 * pw
            cx2[dso] = px + 0.5 * pw
            cy1[dso] = py - 0.5 * ph
            cy2[dso] = py + 0.5 * ph
            w1h1[dso] = pw * ph
            xs[dso] = xv
            ys[dso] = yv
            cfs[dso] = cfv
            bl = _sl1(xv - 0.5) + _sl1(yv - 0.5) + _sl1(wv) + _sl1(hv)
            acc = acc + jnp.where(mreal, 0.5 * bl, 0.0)
        return acc

    acc = lax.fori_loop(0, 12, prep_body, jnp.zeros((16,), jnp.float32))

    def iou_body(i, acc):
        dsi = pl.ds(i * 16, 16)
        a1 = cx1[dsi]
        a2 = cx2[dsi]
        d1 = cy1[dsi]
        d2 = cy2[dsi]
        w1 = a2 - a1
        h1 = d2 - d1
        ar1 = w1h1[dsi]

        def t_body(t, cur_v):
            cw = w1 + sw2[t] - (jnp.maximum(a2, sb2x2[t])
                                - jnp.minimum(a1, sb2x1[t]))
            ch = h1 + sh2[t] - (jnp.maximum(d2, sb2y2[t])
                                - jnp.minimum(d1, sb2y1[t]))
            inter = jnp.where((cw > 0.0) & (ch > 0.0), cw * ch, 0.0)
            return jnp.maximum(cur_v, inter / (ar1 + swh[t] - inter))

        cur_v = lax.fori_loop(0, 50, t_body, jnp.zeros((16,), jnp.float32),
                              unroll=10)
        cur[dsi] = cur_v
        p16 = (i % 12) * 16 + _iota16()
        mreal = p16 >= h * 23
        cf = cfs[dsi]
        acc = acc + jnp.where(mreal & (cur_v <= THRESH), 0.5 * cf * cf, 0.0)
        return acc

    acc = lax.fori_loop(0, 60, iou_body, acc)

    for ci in range(4):
        dsl = pl.ds(16 * ci, 16)
        t16 = _iota16() + 16 * ci
        myidx = idxa[dsl]
        vald = vala[dsl] != 0

        def kill_body(t2, killed):
            hit = (myidx == sidx[t2]) & (t2 > t16) & (sval[t2] != 0)
            return killed | hit.astype(jnp.int32)

        killed = lax.fori_loop(1, 50, kill_body, jnp.zeros((16,), jnp.int32),
                               unroll=10)

        pix = pixa[dsl]
        lower = (pix < HALF).astype(jnp.int32)
        own = (lower * (1 - h) + (1 - lower) * h) == 1
        wr = vald & (killed == 0) & own

        ploc = jnp.clip(pix - start, 0, HALF - 1)
        bn = bna[dsl]
        cell = bn * HALF + ploc

        xc = plsc.load_gather(xs, [cell])
        yc = plsc.load_gather(ys, [cell])
        cfc = plsc.load_gather(cfs, [cell])
        curc = plsc.load_gather(cur, [cell])
        c1c = plsc.load_gather(cx1, [cell])
        c2c = plsc.load_gather(cx2, [cell])
        d1c = plsc.load_gather(cy1, [cell])
        d2c = plsc.load_gather(cy2, [cell])
        arc = plsc.load_gather(w1h1, [cell])
        rowb = (bn * 25) * NPIX + pix
        wc = plsc.load_gather(raw, [rowb + 2 * NPIX])
        hc = plsc.load_gather(raw, [rowb + 3 * NPIX])

        gx = gxr[dsl]
        gy = gyr[dsl]
        gw = gwr[dsl]
        gh = ghr[dsl]
        sx1 = gx - 0.5 * gw
        sx2 = gx + 0.5 * gw
        sy1 = gy - 0.5 * gh
        sy2 = gy + 0.5 * gh
        cw = (c2c - c1c) + gw - (jnp.maximum(c2c, sx2) - jnp.minimum(c1c, sx1))
        chh = (d2c - d1c) + gh - (jnp.maximum(d2c, sy2) - jnp.minimum(d1c, sy1))
        inter = jnp.where((cw > 0.0) & (chh > 0.0), cw * chh, 0.0)
        ioup = inter / (arc + gw * gh - inter)

        logits = [plsc.load_gather(raw, [rowb + (5 + cc) * NPIX])
                  for cc in range(NCLS)]
        mval = logits[0]
        for cc in range(1, NCLS):
            mval = jnp.maximum(mval, logits[cc])
        ssum = jnp.zeros((16,), jnp.float32)
        for cc in range(NCLS):
            ssum = ssum + jnp.exp(logits[cc] - mval)
        lse = mval + _vlog(ssum)
        clsv = clsa[dsl]
        pick = logits[0]
        for cc in range(1, NCLS):
            pick = jnp.where(clsv == cc, logits[cc], pick)
        logp = pick - lse

        txv = txva[dsl]
        tyv = tyva[dsl]
        twv = twva[dsl]
        thv = thva[dsl]
        dx = _sl1(xc - txv) - _sl1(xc - 0.5)
        dy = _sl1(yc - tyv) - _sl1(yc - 0.5)
        dw = _sl1(wc - twv) - _sl1(wc)
        dh = _sl1(hc - thv) - _sl1(hc)
        dcf = (OBJ_SCALE * (cfc - ioup) * (cfc - ioup)
               - jnp.where(curc <= THRESH, cfc * cfc, 0.0))
        corr = 0.5 * (dx + dy + dw + dh + dcf) - logp
        acc = acc + jnp.where(wr, corr, 0.0)

    accv[...] = acc
    pltpu.sync_copy(accv, out.at[wid])


def kernel(output, target):
    out3 = output.reshape(16, 1, 125 * NPIX)
    tgt2 = jnp.pad(target, ((0, 0), (0, 70))).reshape(16, 1, 320)
    parts = _region_loss_sc(out3, tgt2)
    return jnp.sum(parts)


# speedup vs baseline: 21.7223x; 1.0578x over previous
import functools

import jax
import jax.numpy as jnp
from jax import lax
from jax.experimental import pallas as pl
from jax.experimental.pallas import tpu as pltpu
from jax.experimental.pallas import tpu_sc as plsc

NA = 5
NCLS = 20
NHW = 19
NPIX = 361
HALF = 192
OVER = 169
AW = (1.3221, 3.19275, 5.05587, 9.47112, 11.2364)
AH = (1.73145, 4.00944, 8.09892, 4.84053, 10.0071)
THRESH = 0.6
OBJ_SCALE = 5.0
LN2 = 0.6931471805599453
SQRT2 = 1.4142135381698608


def _iota16():
    return lax.iota(jnp.int32, 16)


def _sigmoid(v):
    return 1.0 / (1.0 + jnp.exp(-v))


def _sl1(d):
    a = jnp.abs(d)
    return jnp.where(a < 1.0, 0.5 * a * a, a - 0.5)


def _vlog(v):
    bits = plsc.bitcast(v, jnp.int32)
    e = (bits >> 23) - 127
    m = plsc.bitcast((bits & 0x007FFFFF) | 0x3F800000, jnp.float32)
    big = m >= SQRT2
    m = jnp.where(big, m * 0.5, m)
    ef = e.astype(jnp.float32) + jnp.where(big, 1.0, 0.0)
    s = (m - 1.0) / (m + 1.0)
    z = s * s
    p = s * (2.0 + z * (0.6666666865 + z * (0.4000000059
             + z * (0.2857142985 + z * 0.2222222222))))
    return ef * LN2 + p


def _pick_anchor(bn, vals):
    r = jnp.full((16,), vals[0], jnp.float32)
    for a in range(1, NA):
        r = jnp.where(bn == a, vals[a], r)
    return r


_MESH = plsc.VectorSubcoreMesh(core_axis_name="c", subcore_axis_name="s")

_SCRATCH = [
    pltpu.VMEM((125 * NPIX,), jnp.float32),
    pltpu.VMEM((250,), jnp.float32),
    pltpu.VMEM((960,), jnp.float32),
    pltpu.VMEM((960,), jnp.float32),
    pltpu.VMEM((960,), jnp.float32),
    pltpu.VMEM((960,), jnp.float32),
    pltpu.VMEM((960,), jnp.float32),
    pltpu.VMEM((960,), jnp.float32),
    pltpu.VMEM((960,), jnp.float32),
    pltpu.VMEM((960,), jnp.float32),
    pltpu.VMEM((960,), jnp.float32),
    pltpu.VMEM((64,), jnp.float32),
    pltpu.VMEM((64,), jnp.float32),
    pltpu.VMEM((64,), jnp.float32),
    pltpu.VMEM((64,), jnp.float32),
    pltpu.VMEM((64,), jnp.float32),
    pltpu.VMEM((64,), jnp.float32),
    pltpu.VMEM((64,), jnp.float32),
    pltpu.VMEM((64,), jnp.float32),
    pltpu.VMEM((64,), jnp.int32),
    pltpu.VMEM((64,), jnp.int32),
    pltpu.VMEM((64,), jnp.int32),
    pltpu.VMEM((64,), jnp.int32),
    pltpu.VMEM((64,), jnp.int32),
    pltpu.VMEM((16,), jnp.float32),
    pltpu.SMEM((64,), jnp.float32),
    pltpu.SMEM((64,), jnp.float32),
    pltpu.SMEM((64,), jnp.float32),
    pltpu.SMEM((64,), jnp.float32),
    pltpu.SMEM((64,), jnp.float32),
    pltpu.SMEM((64,), jnp.int32),
    pltpu.SMEM((64,), jnp.int32),
]


@functools.partial(
    pl.kernel,
    out_type=jax.ShapeDtypeStruct((32, 16), jnp.float32),
    mesh=_MESH,
    scratch_types=_SCRATCH,
    compiler_params=pltpu.CompilerParams(needs_layout_passes=False),
)
def _region_loss_sc(out3, tgt2, out,
                    raw, tgt,
                    cx1, cx2, cy1, cy2, w1h1, xs, ys, cfs, over,
                    gxr, gyr, gwr, ghr, txva, tyva, twva, thva,
                    idxa, pixa, vala, bna, clsa, accv,
                    sb2x1, sb2x2, sb2y1, sb2y2, sthr, sidx, sval):
    c = lax.axis_index("c")
    s = lax.axis_index("s")
    b = s
    h = c
    start = h * OVER
    wid = s * 2 + c

    pltpu.sync_copy(tgt2.at[b, 0], tgt)
    pltpu.sync_copy(out3.at[b, 0], raw)

    nzeros = jnp.int32(0)
    for ci in range(4):
        t16 = _iota16() + 16 * ci
        base5 = jnp.minimum(t16, 49) * 5
        tc0 = plsc.load_gather(tgt, [base5])
        gx = plsc.load_gather(tgt, [base5 + 1]) * float(NHW)
        gy = plsc.load_gather(tgt, [base5 + 2]) * float(NHW)
        gw = plsc.load_gather(tgt, [base5 + 3]) * float(NHW)
        gh = plsc.load_gather(tgt, [base5 + 4]) * float(NHW)

        zc = jnp.where((t16 < 50) & (gx != 0.0), 0, 1).astype(jnp.int32)
        cz = plsc.cumsum(zc)
        vald = (cz + nzeros) == 0
        nzeros = nzeros + jnp.sum(zc)

        best_iou = jnp.full((16,), -1.0, jnp.float32)
        bn = jnp.zeros((16,), jnp.int32)
        for a in range(NA):
            ca = jnp.minimum(gw, AW[a]) * jnp.minimum(gh, AH[a])
            ai = ca / (AW[a] * AH[a] + gw * gh - ca)
            better = ai > best_iou
            best_iou = jnp.where(better, ai, best_iou)
            bn = jnp.where(better, a, bn)

        gi = gx.astype(jnp.int32)
        gj = gy.astype(jnp.int32)
        awn = _pick_anchor(bn, AW)
        ahn = _pick_anchor(bn, AH)
        pix = gj * NHW + gi

        gws = jnp.where(vald, gw, 0.0)
        ghs = jnp.where(vald, gh, 0.0)
        vx1 = gx - 0.5 * gws
        vx2 = gx + 0.5 * gws
        vy1 = gy - 0.5 * ghs
        vy2 = gy + 0.5 * ghs
        vth = 0.375 * (gws * ghs)
        vidx = bn * NPIX + pix
        vvalI = vald.astype(jnp.int32)

        dsl = pl.ds(16 * ci, 16)
        gxr[dsl] = gx
        gyr[dsl] = gy
        gwr[dsl] = gw
        ghr[dsl] = gh
        txva[dsl] = gx - gi.astype(jnp.float32)
        tyva[dsl] = gy - gj.astype(jnp.float32)
        twva[dsl] = _vlog(gw / awn)
        thva[dsl] = _vlog(gh / ahn)
        idxa[dsl] = vidx
        pixa[dsl] = pix
        vala[dsl] = vvalI
        bna[dsl] = bn
        clsa[dsl] = tc0.astype(jnp.int32)

        for k in range(16):
            t = 16 * ci + k
            sb2x1[t] = vx1[k]
            sb2x2[t] = vx2[k]
            sb2y1[t] = vy1[k]
            sb2y2[t] = vy2[k]
            sthr[t] = vth[k]
            sidx[t] = vidx[k]
            sval[t] = vvalI[k]

    def prep_body(j, acc):
        p16 = j * 16 + _iota16()
        pg = p16 + start
        gxg = (pg % NHW).astype(jnp.float32)
        gyg = (pg // NHW).astype(jnp.float32)
        mreal = p16 >= h * 23
        col = start + j * 16
        for a in range(NA):
            r0 = a * 25
            xv = _sigmoid(raw[pl.ds((r0 + 0) * NPIX + col, 16)])
            yv = _sigmoid(raw[pl.ds((r0 + 1) * NPIX + col, 16)])
            wv = raw[pl.ds((r0 + 2) * NPIX + col, 16)]
            hv = raw[pl.ds((r0 + 3) * NPIX + col, 16)]
            cfv = _sigmoid(raw[pl.ds((r0 + 4) * NPIX + col, 16)])
            px = xv + gxg
            py = yv + gyg
            pw = jnp.exp(wv) * AW[a]
            ph = jnp.exp(hv) * AH[a]
            dso = pl.ds(a * HALF + j * 16, 16)
            cx1[dso] = px - 0.5 * pw
            cx2[dso] = px + 0.5 * pw
            cy1[dso] = py - 0.5 * ph
            cy2[dso] = py + 0.5 * ph
            w1h1[dso] = pw * ph
            xs[dso] = xv
            ys[dso] = yv
            cfs[dso] = cfv
            bl = _sl1(xv - 0.5) + _sl1(yv - 0.5) + _sl1(wv) + _sl1(hv)
            acc = acc + jnp.where(mreal, 0.5 * bl, 0.0)
        return acc

    acc = lax.fori_loop(0, 12, prep_body, jnp.zeros((16,), jnp.float32))

    def iou_body(ib, acc):
        a1q = []
        a2q = []
        d1q = []
        d2q = []
        thq = []
        for q in range(4):
            dsq = pl.ds(ib * 64 + q * 16, 16)
            a1q.append(cx1[dsq])
            a2q.append(cx2[dsq])
            d1q.append(cy1[dsq])
            d2q.append(cy2[dsq])
            thq.append(0.375 * w1h1[dsq])

        def t_body(t, ovs):
            sx1 = sb2x1[t]
            sx2 = sb2x2[t]
            sy1 = sb2y1[t]
            sy2 = sb2y2[t]
            st = sthr[t]
            new = []
            for q in range(4):
                cw = jnp.minimum(a2q[q], sx2) - jnp.maximum(a1q[q], sx1)
                ch = jnp.minimum(d2q[q], sy2) - jnp.maximum(d1q[q], sy1)
                hit = (cw > 0.0) & (ch > 0.0) & (cw * ch > thq[q] + st)
                new.append(ovs[q] | hit.astype(jnp.int32))
            return tuple(new)

        z16 = jnp.zeros((16,), jnp.int32)
        ovs = lax.fori_loop(0, 50, t_body, (z16, z16, z16, z16), unroll=5)
        for q in range(4):
            i = ib * 4 + q
            dsq = pl.ds(i * 16, 16)
            over[dsq] = ovs[q].astype(jnp.float32)
            p16 = (i % 12) * 16 + _iota16()
            mreal = p16 >= h * 23
            cf = cfs[dsq]
            acc = acc + jnp.where(mreal & (ovs[q] == 0), 0.5 * cf * cf, 0.0)
        return acc

    acc = lax.fori_loop(0, 15, iou_body, acc)

    for ci in range(4):
        dsl = pl.ds(16 * ci, 16)
        t16 = _iota16() + 16 * ci
        myidx = idxa[dsl]
        vald = vala[dsl] != 0

        def kill_body(t2, killed):
            hit = (myidx == sidx[t2]) & (t2 > t16) & (sval[t2] != 0)
            return killed | hit.astype(jnp.int32)

        killed = lax.fori_loop(1, 50, kill_body, jnp.zeros((16,), jnp.int32),
                               unroll=10)

        pix = pixa[dsl]
        lower = (pix < HALF).astype(jnp.int32)
        own = (lower * (1 - h) + (1 - lower) * h) == 1
        wr = vald & (killed == 0) & own

        ploc = jnp.clip(pix - start, 0, HALF - 1)
        bn = bna[dsl]
        cell = bn * HALF + ploc

        xc = plsc.load_gather(xs, [cell])
        yc = plsc.load_gather(ys, [cell])
        cfc = plsc.load_gather(cfs, [cell])
        ovc = plsc.load_gather(over, [cell])
        c1c = plsc.load_gather(cx1, [cell])
        c2c = plsc.load_gather(cx2, [cell])
        d1c = plsc.load_gather(cy1, [cell])
        d2c = plsc.load_gather(cy2, [cell])
        arc = plsc.load_gather(w1h1, [cell])
        rowb = (bn * 25) * NPIX + pix
        wc = plsc.load_gather(raw, [rowb + 2 * NPIX])
        hc = plsc.load_gather(raw, [rowb + 3 * NPIX])

        gx = gxr[dsl]
        gy = gyr[dsl]
        gw = gwr[dsl]
        gh = ghr[dsl]
        sx1 = gx - 0.5 * gw
        sx2 = gx + 0.5 * gw
        sy1 = gy - 0.5 * gh
        sy2 = gy + 0.5 * gh
        cw = (c2c - c1c) + gw - (jnp.maximum(c2c, sx2) - jnp.minimum(c1c, sx1))
        chh = (d2c - d1c) + gh - (jnp.maximum(d2c, sy2) - jnp.minimum(d1c, sy1))
        inter = jnp.where((cw > 0.0) & (chh > 0.0), cw * chh, 0.0)
        ioup = inter / (arc + gw * gh - inter)

        logits = [plsc.load_gather(raw, [rowb + (5 + cc) * NPIX])
                  for cc in range(NCLS)]
        mval = logits[0]
        for cc in range(1, NCLS):
            mval = jnp.maximum(mval, logits[cc])
        ssum = jnp.zeros((16,), jnp.float32)
        for cc in range(NCLS):
            ssum = ssum + jnp.exp(logits[cc] - mval)
        lse = mval + _vlog(ssum)
        clsv = clsa[dsl]
        pick = logits[0]
        for cc in range(1, NCLS):
            pick = jnp.where(clsv == cc, logits[cc], pick)
        logp = pick - lse

        txv = txva[dsl]
        tyv = tyva[dsl]
        twv = twva[dsl]
        thv = thva[dsl]
        dx = _sl1(xc - txv) - _sl1(xc - 0.5)
        dy = _sl1(yc - tyv) - _sl1(yc - 0.5)
        dw = _sl1(wc - twv) - _sl1(wc)
        dh = _sl1(hc - thv) - _sl1(hc)
        dcf = (OBJ_SCALE * (cfc - ioup) * (cfc - ioup)
               - jnp.where(ovc == 0.0, cfc * cfc, 0.0))
        corr = 0.5 * (dx + dy + dw + dh + dcf) - logp
        acc = acc + jnp.where(wr, corr, 0.0)

    accv[...] = acc
    pltpu.sync_copy(accv, out.at[wid])


def kernel(output, target):
    out3 = output.reshape(16, 1, 125 * NPIX)
    tgt2 = target.reshape(16, 1, 250)
    parts = _region_loss_sc(out3, tgt2)
    return jnp.sum(parts)
